# baseline (speedup 1.0000x reference)
import functools

import jax
import jax.numpy as jnp
from jax import lax
from jax.experimental import pallas as pl
from jax.experimental.pallas import tpu as pltpu
from jax.experimental.pallas import tpu_sc as plsc

_BATCH = 16384
_HIST = 200
_D = 32
_N = _BATCH * _HIST

_info = plsc.get_sparse_core_info()
_NC, _NS = _info.num_cores, _info.num_subcores
_NW = _NC * _NS
_PER_W = _N // _NW
_C = 800
_NB = 4
_NCHUNK = _PER_W // _C


def _make_kernel():
    mesh = plsc.VectorSubcoreMesh(core_axis_name="c", subcore_axis_name="s")

    @functools.partial(
        pl.kernel,
        mesh=mesh,
        out_type=jax.ShapeDtypeStruct((_N, _D), jnp.float32),
        scratch_types=[
            pltpu.VMEM((_NB, _C), jnp.int32),
            pltpu.VMEM((_NB, _C, _D), jnp.float32),
            pltpu.SemaphoreType.DMA((_NB,)),
            pltpu.SemaphoreType.DMA((_NB,)),
            pltpu.SemaphoreType.DMA((_NB,)),
        ],
        compiler_params=pltpu.CompilerParams(
            use_tc_tiling_on_sc=False, needs_layout_passes=False
        ),
    )
    def body(idx_hbm, w_hbm, out_hbm, idx_v, rows_v, sem_i, sem_g, sem_o):
        wid = lax.axis_index("s") * _NC + lax.axis_index("c")
        r0 = wid * _PER_W

        def start_idx(g, par):
            pltpu.async_copy(
                idx_hbm.at[pl.ds(r0 + g * _C, _C)], idx_v.at[par],
                sem_i.at[par],
            )

        def wait_idx(par):
            pltpu.make_async_copy(
                idx_hbm.at[pl.ds(0, _C)], idx_v.at[par], sem_i.at[par]
            ).wait()

        def start_gather(par):
            pltpu.async_copy(
                w_hbm.at[idx_v.at[par]], rows_v.at[par], sem_g.at[par]
            )

        def wait_gather(par):
            pltpu.make_async_copy(
                w_hbm.at[pl.ds(0, _C)], rows_v.at[par], sem_g.at[par]
            ).wait()

        def start_store(g, par):
            pltpu.async_copy(
                rows_v.at[par], out_hbm.at[pl.ds(r0 + g * _C, _C)],
                sem_o.at[par],
            )

        def wait_store(par):
            pltpu.make_async_copy(
                rows_v.at[par], out_hbm.at[pl.ds(0, _C)], sem_o.at[par]
            ).wait()

        def step(g, p):
            q = (p - 2) % _NB
            wait_idx(p)

            @pl.when(g >= _NB)
            def _():
                wait_store(p)

            start_gather(p)

            @pl.when(g >= 2)
            def _():
                wait_gather(q)
                start_store(g - 2, q)

            @pl.when((g >= 2) & (g + _NB - 2 < _NCHUNK))
            def _():
                start_idx(g + _NB - 2, q)

        for b in range(_NB):
            start_idx(b, b)
        wait_idx(0)
        start_gather(0)

        def quad(qd, carry):
            g = _NB * qd + 1
            for k in range(_NB):
                step(g + k, (1 + k) % _NB)
            return carry

        lax.fori_loop(0, (_NCHUNK - 1) // _NB, quad, 0)

        for g in range(1 + _NB * ((_NCHUNK - 1) // _NB), _NCHUNK):
            step(g, g % _NB)
        for g in (_NCHUNK - 2, _NCHUNK - 1):
            b = g % _NB
            wait_gather(b)
            start_store(g, b)
        for b in range(_NB):
            wait_store(b)

    return body


_gather_call = _make_kernel()


def kernel(input_tensor, weight):
    out_flat = _gather_call(input_tensor.reshape(_N), weight)
    return out_flat.reshape(_BATCH, _HIST, _D)


# speedup vs baseline: 1.0004x; 1.0004x over previous
import functools

import jax
import jax.numpy as jnp
from jax import lax
from jax.experimental import pallas as pl
from jax.experimental.pallas import tpu as pltpu
from jax.experimental.pallas import tpu_sc as plsc

_BATCH = 16384
_HIST = 200
_D = 32
_N = _BATCH * _HIST

_info = plsc.get_sparse_core_info()
_NC, _NS = _info.num_cores, _info.num_subcores
_NW = _NC * _NS
_PER_W = _N // _NW
_C = 800
_NB = 4
_NCHUNK = _PER_W // _C


def _make_kernel():
    mesh = plsc.VectorSubcoreMesh(core_axis_name="c", subcore_axis_name="s")

    @functools.partial(
        pl.kernel,
        mesh=mesh,
        out_type=jax.ShapeDtypeStruct((_N, _D), jnp.float32),
        scratch_types=[
            pltpu.VMEM((_NB, _C), jnp.int32),
            pltpu.VMEM((_NB, _C, _D), jnp.float32),
            pltpu.SemaphoreType.DMA((_NB,)),
            pltpu.SemaphoreType.DMA((_NB,)),
            pltpu.SemaphoreType.DMA((_NB,)),
        ],
        compiler_params=pltpu.CompilerParams(
            use_tc_tiling_on_sc=False, needs_layout_passes=False
        ),
    )
    def body(idx_hbm, w_hbm, out_hbm, idx_v, rows_v, sem_i, sem_g, sem_o):
        wid = lax.axis_index("s") * _NC + lax.axis_index("c")
        r0 = wid * _PER_W

        def start_idx(g, par):
            pltpu.async_copy(
                idx_hbm.at[pl.ds(r0 + g * _C, _C)], idx_v.at[par],
                sem_i.at[par],
            )

        def wait_idx(par):
            pltpu.make_async_copy(
                idx_hbm.at[pl.ds(0, _C)], idx_v.at[par], sem_i.at[par]
            ).wait()

        def start_gather(par):
            pltpu.async_copy(
                w_hbm.at[idx_v.at[par]], rows_v.at[par], sem_g.at[par]
            )

        def wait_gather(par):
            pltpu.make_async_copy(
                w_hbm.at[pl.ds(0, _C)], rows_v.at[par], sem_g.at[par]
            ).wait()

        def start_store(g, par):
            pltpu.async_copy(
                rows_v.at[par], out_hbm.at[pl.ds(r0 + g * _C, _C)],
                sem_o.at[par],
            )

        def wait_store(par):
            pltpu.make_async_copy(
                rows_v.at[par], out_hbm.at[pl.ds(0, _C)], sem_o.at[par]
            ).wait()

        def step(g, p):
            q = (p - 1) % _NB
            wait_idx(p)

            @pl.when(g >= _NB)
            def _():
                wait_store(p)

            start_gather(p)
            wait_gather(q)
            start_store(g - 1, q)

            @pl.when(g + _NB - 1 < _NCHUNK)
            def _():
                start_idx(g + _NB - 1, q)

        for b in range(_NB):
            start_idx(b, b)
        wait_idx(0)
        start_gather(0)

        def quad(qd, carry):
            g = _NB * qd + 1
            for k in range(_NB):
                step(g + k, (g + k) % _NB)
            return carry

        lax.fori_loop(0, (_NCHUNK - 1) // _NB, quad, 0)

        for g in range(1 + _NB * ((_NCHUNK - 1) // _NB), _NCHUNK):
            step(g, g % _NB)
        last = (_NCHUNK - 1) % _NB
        wait_gather(last)
        start_store(_NCHUNK - 1, last)
        for b in range(_NB):
            wait_store(b)

    return body


_gather_call = _make_kernel()


def kernel(input_tensor, weight):
    out_flat = _gather_call(input_tensor.reshape(_N), weight)
    return out_flat.reshape(_BATCH, _HIST, _D)
